# baseline (speedup 1.0000x reference)
import jax
import jax.numpy as jnp
from jax import lax
from jax.experimental import pallas as pl
from jax.experimental.pallas import tpu as pltpu

NEG_SLOPE = 0.2
BN_EPS = 1e-5
BB = 8


def _leaky(x):
    return jnp.where(x >= 0, x, NEG_SLOPE * x)


def _kern_a(x_ref, wcat_ref, btr_ref, adjn_ref, ws_ref, wg_ref,
            out1_ref, vt_ref, h_ref, y_ref, t_scr, *, n_nodes, n_pad, d):
    for i in range(BB):
        x_bf = x_ref[i].astype(jnp.bfloat16)
        s = lax.dot_general(x_bf, wcat_ref[...],
                            (((0,), (1,)), ((), ())),
                            preferred_element_type=jnp.float32)
        sc = s[:, :n_pad]
        out1_ref[i:i + 1, :] = jnp.max(sc[:, :n_nodes], axis=0,
                                       keepdims=True)
        mask = jax.nn.sigmoid(sc)
        xt = s[:, n_pad:] + btr_ref[...]
        vt_i = lax.dot_general(mask.astype(jnp.bfloat16),
                               xt.astype(jnp.bfloat16),
                               (((0,), (0,)), ((), ())),
                               preferred_element_type=jnp.float32)
        vt_ref[i * n_nodes:(i + 1) * n_nodes, :] = \
            vt_i[:n_nodes, :].astype(jnp.bfloat16)
    v_bf = vt_ref[...]
    for i in range(BB):
        t_i = jnp.dot(adjn_ref[...],
                      v_bf[i * n_nodes:(i + 1) * n_nodes, :],
                      preferred_element_type=jnp.float32)
        t_scr[i * n_nodes:(i + 1) * n_nodes, :] = \
            _leaky(t_i).astype(jnp.bfloat16)
    h_all = v_bf.astype(jnp.float32) + jnp.dot(
        t_scr[...], ws_ref[...], preferred_element_type=jnp.float32)
    h_ref[...] = h_all.astype(jnp.bfloat16)
    xglb = jnp.mean(h_all.reshape(BB, n_nodes, d), axis=1)
    y_ref[...] = lax.dot_general(xglb.astype(jnp.bfloat16), wg_ref[...],
                                 (((1,), (1,)), ((), ())),
                                 preferred_element_type=jnp.float32)


def _kern_b(h_ref, vt_ref, g_ref, wcog_ref, wcox_ref, bco_ref, sadj_ref,
            wdyn_ref, wlast_ref, blast_ref, out2_ref, t_scr, *, n_nodes, d):
    tg = lax.dot_general(wcog_ref[...], g_ref[...].astype(jnp.bfloat16),
                         (((1,), (1,)), ((), ())),
                         preferred_element_type=jnp.float32)
    for i in range(BB):
        h_bf = h_ref[i * n_nodes:(i + 1) * n_nodes, :]
        tx = lax.dot_general(wcox_ref[...], h_bf,
                             (((1,), (1,)), ((), ())),
                             preferred_element_type=jnp.float32)
        a = jax.nn.sigmoid(tx + tg[:, i:i + 1] + bco_ref[...])
        a = (a + sadj_ref[...]) * 0.5
        dv = lax.rsqrt(jnp.sum(a, axis=1, keepdims=True))
        m = (dv * h_bf.astype(jnp.float32)).astype(jnp.bfloat16)
        t_i = lax.dot_general(a.astype(jnp.bfloat16), m,
                              (((0,), (0,)), ((), ())),
                              preferred_element_type=jnp.float32)
        t_scr[i * n_nodes:(i + 1) * n_nodes, :] = \
            _leaky(dv * t_i).astype(jnp.bfloat16)
    z = _leaky(jnp.dot(t_scr[...], wdyn_ref[...],
                       preferred_element_type=jnp.float32))
    y = vt_ref[...].astype(jnp.float32) + z
    yw = y.reshape(BB, n_nodes, d) * wlast_ref[...][None]
    out2_ref[...] = jnp.sum(yw, axis=2) + blast_ref[...]


def kernel(x_feat, static_adj, static_weight, dynamic_weight, w_fc, w_tr,
           b_tr, w_g, b_g, bn_gamma, bn_beta, w_co, b_co, w_last, b_last):
    B, Cf, H, W = x_feat.shape
    n_nodes = w_fc.shape[0]
    d = w_tr.shape[0]
    n_pad = 128
    nblk = B // BB
    hw = H * W

    xr = x_feat.reshape(B, Cf, hw)
    wcat_bf = jnp.concatenate(
        [w_fc, jnp.zeros((n_pad - n_nodes, Cf), jnp.float32), w_tr],
        axis=0).astype(jnp.bfloat16)
    A = static_adj
    dvec = jnp.sum(A, axis=1) ** -0.5
    adjn = (dvec[:, None] * A.T * dvec[None, :]).astype(jnp.bfloat16)

    out1, vt, h, y = pl.pallas_call(
        lambda *refs: _kern_a(*refs, n_nodes=n_nodes, n_pad=n_pad, d=d),
        grid=(nblk,),
        in_specs=[
            pl.BlockSpec((BB, Cf, hw), lambda i: (i, 0, 0)),
            pl.BlockSpec((n_pad + d, Cf), lambda i: (0, 0)),
            pl.BlockSpec((1, d), lambda i: (0, 0)),
            pl.BlockSpec((n_nodes, n_nodes), lambda i: (0, 0)),
            pl.BlockSpec((d, d), lambda i: (0, 0)),
            pl.BlockSpec((d, d), lambda i: (0, 0)),
        ],
        out_specs=[
            pl.BlockSpec((BB, n_nodes), lambda i: (i, 0)),
            pl.BlockSpec((BB * n_nodes, d), lambda i: (i, 0)),
            pl.BlockSpec((BB * n_nodes, d), lambda i: (i, 0)),
            pl.BlockSpec((BB, d), lambda i: (i, 0)),
        ],
        out_shape=[
            jax.ShapeDtypeStruct((B, n_nodes), jnp.float32),
            jax.ShapeDtypeStruct((B * n_nodes, d), jnp.bfloat16),
            jax.ShapeDtypeStruct((B * n_nodes, d), jnp.bfloat16),
            jax.ShapeDtypeStruct((B, d), jnp.float32),
        ],
        scratch_shapes=[pltpu.VMEM((BB * n_nodes, d), jnp.bfloat16)],
        compiler_params=pltpu.CompilerParams(
            dimension_semantics=("parallel",)),
    )(xr, wcat_bf, b_tr.reshape(1, d), adjn, static_weight.astype(jnp.bfloat16),
      w_g.astype(jnp.bfloat16))

    y = y + b_g
    mu = jnp.mean(y, axis=0, keepdims=True)
    var = jnp.mean((y - mu) ** 2, axis=0, keepdims=True)
    g = _leaky((y - mu) * lax.rsqrt(var + BN_EPS) * bn_gamma + bn_beta)

    out2 = pl.pallas_call(
        lambda *refs: _kern_b(*refs, n_nodes=n_nodes, d=d),
        grid=(nblk,),
        in_specs=[
            pl.BlockSpec((BB * n_nodes, d), lambda i: (i, 0)),
            pl.BlockSpec((BB * n_nodes, d), lambda i: (i, 0)),
            pl.BlockSpec((BB, d), lambda i: (i, 0)),
            pl.BlockSpec((n_nodes, d), lambda i: (0, 0)),
            pl.BlockSpec((n_nodes, d), lambda i: (0, 0)),
            pl.BlockSpec((n_nodes, 1), lambda i: (0, 0)),
            pl.BlockSpec((n_nodes, n_nodes), lambda i: (0, 0)),
            pl.BlockSpec((d, d), lambda i: (0, 0)),
            pl.BlockSpec((n_nodes, d), lambda i: (0, 0)),
            pl.BlockSpec((1, n_nodes), lambda i: (0, 0)),
        ],
        out_specs=pl.BlockSpec((BB, n_nodes), lambda i: (i, 0)),
        out_shape=jax.ShapeDtypeStruct((B, n_nodes), jnp.float32),
        scratch_shapes=[pltpu.VMEM((BB * n_nodes, d), jnp.bfloat16)],
        compiler_params=pltpu.CompilerParams(
            dimension_semantics=("parallel",)),
    )(h, vt, g, w_co[:, :d].astype(jnp.bfloat16),
      w_co[:, d:].astype(jnp.bfloat16), b_co, static_adj,
      dynamic_weight.astype(jnp.bfloat16), w_last, b_last.reshape(1, n_nodes))
    return out1, out2


# speedup vs baseline: 1.5236x; 1.5236x over previous
import jax
import jax.numpy as jnp
from jax import lax
from jax.experimental import pallas as pl
from jax.experimental.pallas import tpu as pltpu

NEG_SLOPE = 0.2
BN_EPS = 1e-5
BB = 8


def _leaky(x):
    return jnp.where(x >= 0, x, NEG_SLOPE * x)


def _kern_a(x_ref, wfc_ref, wtr_ref, btr_ref, adjn_ref, ws_ref, wg_ref,
            out1_ref, vt_ref, h_ref, y_ref, t_scr, *, n_nodes, d, hw):
    sc_all = jnp.dot(x_ref[...], wfc_ref[...],
                     preferred_element_type=jnp.float32)
    xt_all = jnp.dot(x_ref[...], wtr_ref[...],
                     preferred_element_type=jnp.float32) + btr_ref[...]
    for i in range(BB):
        sc = sc_all[i * hw:(i + 1) * hw, :]
        out1_ref[i:i + 1, :] = jnp.max(sc, axis=0, keepdims=True)
        mask = jax.nn.sigmoid(sc)
        xt = xt_all[i * hw:(i + 1) * hw, :]
        vt_i = lax.dot_general(mask.astype(jnp.bfloat16),
                               xt.astype(jnp.bfloat16),
                               (((0,), (0,)), ((), ())),
                               preferred_element_type=jnp.float32)
        vt_ref[i * n_nodes:(i + 1) * n_nodes, :] = \
            vt_i.astype(jnp.bfloat16)
    v_bf = vt_ref[...]
    for i in range(BB):
        t_i = jnp.dot(adjn_ref[...],
                      v_bf[i * n_nodes:(i + 1) * n_nodes, :],
                      preferred_element_type=jnp.float32)
        t_scr[i * n_nodes:(i + 1) * n_nodes, :] = \
            _leaky(t_i).astype(jnp.bfloat16)
    h_all = v_bf.astype(jnp.float32) + jnp.dot(
        t_scr[...], ws_ref[...], preferred_element_type=jnp.float32)
    h_ref[...] = h_all.astype(jnp.bfloat16)
    xglb = jnp.mean(h_all.reshape(BB, n_nodes, d), axis=1)
    y_ref[...] = lax.dot_general(xglb.astype(jnp.bfloat16), wg_ref[...],
                                 (((1,), (1,)), ((), ())),
                                 preferred_element_type=jnp.float32)


def _kern_b(h_ref, vt_ref, g_ref, wcog_ref, wcox_ref, bco_ref, sadj_ref,
            wdyn_ref, wlast_ref, blast_ref, out2_ref, t_scr, *, n_nodes, d):
    tg = lax.dot_general(wcog_ref[...], g_ref[...].astype(jnp.bfloat16),
                         (((1,), (1,)), ((), ())),
                         preferred_element_type=jnp.float32)
    for i in range(BB):
        h_bf = h_ref[i * n_nodes:(i + 1) * n_nodes, :]
        tx = lax.dot_general(wcox_ref[...], h_bf,
                             (((1,), (1,)), ((), ())),
                             preferred_element_type=jnp.float32)
        a = jax.nn.sigmoid(tx + tg[:, i:i + 1] + bco_ref[...])
        a = (a + sadj_ref[...]) * 0.5
        dv = lax.rsqrt(jnp.sum(a, axis=1, keepdims=True))
        m = (dv * h_bf.astype(jnp.float32)).astype(jnp.bfloat16)
        t_i = lax.dot_general(a.astype(jnp.bfloat16), m,
                              (((0,), (0,)), ((), ())),
                              preferred_element_type=jnp.float32)
        t_scr[i * n_nodes:(i + 1) * n_nodes, :] = \
            _leaky(dv * t_i).astype(jnp.bfloat16)
    z = _leaky(jnp.dot(t_scr[...], wdyn_ref[...],
                       preferred_element_type=jnp.float32))
    y = vt_ref[...].astype(jnp.float32) + z
    yw = y.reshape(BB, n_nodes, d) * wlast_ref[...][None]
    out2_ref[...] = jnp.sum(yw, axis=2) + blast_ref[...]


def kernel(x_feat, static_adj, static_weight, dynamic_weight, w_fc, w_tr,
           b_tr, w_g, b_g, bn_gamma, bn_beta, w_co, b_co, w_last, b_last):
    B, Cf, H, W = x_feat.shape
    n_nodes = w_fc.shape[0]
    d = w_tr.shape[0]
    nblk = B // BB
    hw = H * W

    xt2 = x_feat.reshape(B, Cf, hw).astype(jnp.bfloat16)
    xt2 = xt2.transpose(0, 2, 1).reshape(B * hw, Cf)
    wfc_t = w_fc.T.astype(jnp.bfloat16)
    wtr_t = w_tr.T.astype(jnp.bfloat16)
    A = static_adj
    dvec = jnp.sum(A, axis=1) ** -0.5
    adjn = (dvec[:, None] * A.T * dvec[None, :]).astype(jnp.bfloat16)

    out1, vt, h, y = pl.pallas_call(
        lambda *refs: _kern_a(*refs, n_nodes=n_nodes, d=d, hw=hw),
        grid=(nblk,),
        in_specs=[
            pl.BlockSpec((BB * hw, Cf), lambda i: (i, 0)),
            pl.BlockSpec((Cf, n_nodes), lambda i: (0, 0)),
            pl.BlockSpec((Cf, d), lambda i: (0, 0)),
            pl.BlockSpec((1, d), lambda i: (0, 0)),
            pl.BlockSpec((n_nodes, n_nodes), lambda i: (0, 0)),
            pl.BlockSpec((d, d), lambda i: (0, 0)),
            pl.BlockSpec((d, d), lambda i: (0, 0)),
        ],
        out_specs=[
            pl.BlockSpec((BB, n_nodes), lambda i: (i, 0)),
            pl.BlockSpec((BB * n_nodes, d), lambda i: (i, 0)),
            pl.BlockSpec((BB * n_nodes, d), lambda i: (i, 0)),
            pl.BlockSpec((BB, d), lambda i: (i, 0)),
        ],
        out_shape=[
            jax.ShapeDtypeStruct((B, n_nodes), jnp.float32),
            jax.ShapeDtypeStruct((B * n_nodes, d), jnp.bfloat16),
            jax.ShapeDtypeStruct((B * n_nodes, d), jnp.bfloat16),
            jax.ShapeDtypeStruct((B, d), jnp.float32),
        ],
        scratch_shapes=[pltpu.VMEM((BB * n_nodes, d), jnp.bfloat16)],
        compiler_params=pltpu.CompilerParams(
            dimension_semantics=("parallel",)),
    )(xt2, wfc_t, wtr_t, b_tr.reshape(1, d), adjn,
      static_weight.astype(jnp.bfloat16), w_g.astype(jnp.bfloat16))

    y = y + b_g
    mu = jnp.mean(y, axis=0, keepdims=True)
    var = jnp.mean((y - mu) ** 2, axis=0, keepdims=True)
    g = _leaky((y - mu) * lax.rsqrt(var + BN_EPS) * bn_gamma + bn_beta)

    out2 = pl.pallas_call(
        lambda *refs: _kern_b(*refs, n_nodes=n_nodes, d=d),
        grid=(nblk,),
        in_specs=[
            pl.BlockSpec((BB * n_nodes, d), lambda i: (i, 0)),
            pl.BlockSpec((BB * n_nodes, d), lambda i: (i, 0)),
            pl.BlockSpec((BB, d), lambda i: (i, 0)),
            pl.BlockSpec((n_nodes, d), lambda i: (0, 0)),
            pl.BlockSpec((n_nodes, d), lambda i: (0, 0)),
            pl.BlockSpec((n_nodes, 1), lambda i: (0, 0)),
            pl.BlockSpec((n_nodes, n_nodes), lambda i: (0, 0)),
            pl.BlockSpec((d, d), lambda i: (0, 0)),
            pl.BlockSpec((n_nodes, d), lambda i: (0, 0)),
            pl.BlockSpec((1, n_nodes), lambda i: (0, 0)),
        ],
        out_specs=pl.BlockSpec((BB, n_nodes), lambda i: (i, 0)),
        out_shape=jax.ShapeDtypeStruct((B, n_nodes), jnp.float32),
        scratch_shapes=[pltpu.VMEM((BB * n_nodes, d), jnp.bfloat16)],
        compiler_params=pltpu.CompilerParams(
            dimension_semantics=("parallel",)),
    )(h, vt, g, w_co[:, :d].astype(jnp.bfloat16),
      w_co[:, d:].astype(jnp.bfloat16), b_co, static_adj,
      dynamic_weight.astype(jnp.bfloat16), w_last, b_last.reshape(1, n_nodes))
    return out1, out2


# speedup vs baseline: 1.6127x; 1.0585x over previous
import jax
import jax.numpy as jnp
from jax import lax
from jax.experimental import pallas as pl
from jax.experimental.pallas import tpu as pltpu

NEG_SLOPE = 0.2
BN_EPS = 1e-5
BB = 16


def _leaky(x):
    return jnp.where(x >= 0, x, NEG_SLOPE * x)


def _kern_a(x_ref, wfc_ref, wtr_ref, btr_ref, adjn_ref, ws_ref, wg_ref,
            wlast_ref, out1_ref, h_ref, y_ref, o2a_ref, vt_scr, t_scr,
            *, n_nodes, d, hw):
    sc_all = jnp.dot(x_ref[...], wfc_ref[...],
                     preferred_element_type=jnp.float32)
    xt_all = jnp.dot(x_ref[...], wtr_ref[...],
                     preferred_element_type=jnp.float32) + btr_ref[...]
    for i in range(BB):
        sc = sc_all[i * hw:(i + 1) * hw, :]
        out1_ref[i:i + 1, :] = jnp.max(sc, axis=0, keepdims=True)
        mask = jax.nn.sigmoid(sc)
        xt = xt_all[i * hw:(i + 1) * hw, :]
        vt_i = lax.dot_general(mask.astype(jnp.bfloat16),
                               xt.astype(jnp.bfloat16),
                               (((0,), (0,)), ((), ())),
                               preferred_element_type=jnp.float32)
        vt_scr[i * n_nodes:(i + 1) * n_nodes, :] = \
            vt_i.astype(jnp.bfloat16)
        o2a_ref[i * n_nodes:(i + 1) * n_nodes, :] = \
            jnp.sum(wlast_ref[...] * vt_i, axis=1, keepdims=True)
    v_bf = vt_scr[...]
    for i in range(BB):
        t_i = jnp.dot(adjn_ref[...],
                      v_bf[i * n_nodes:(i + 1) * n_nodes, :],
                      preferred_element_type=jnp.float32)
        t_scr[i * n_nodes:(i + 1) * n_nodes, :] = \
            _leaky(t_i).astype(jnp.bfloat16)
    h_all = v_bf.astype(jnp.float32) + jnp.dot(
        t_scr[...], ws_ref[...], preferred_element_type=jnp.float32)
    h_ref[...] = h_all.astype(jnp.bfloat16)
    xglb = jnp.mean(h_all.reshape(BB, n_nodes, d), axis=1)
    y_ref[...] = lax.dot_general(xglb.astype(jnp.bfloat16), wg_ref[...],
                                 (((1,), (1,)), ((), ())),
                                 preferred_element_type=jnp.float32)


def _kern_b(h_ref, g_ref, wcog_ref, wcox_ref, bco_ref, sadj_ref,
            wdyn_ref, wlast_ref, out2_ref, t_scr, *, n_nodes, d):
    tg = lax.dot_general(wcog_ref[...], g_ref[...].astype(jnp.bfloat16),
                         (((1,), (1,)), ((), ())),
                         preferred_element_type=jnp.float32)
    for i in range(BB):
        h_bf = h_ref[i * n_nodes:(i + 1) * n_nodes, :]
        tx = lax.dot_general(wcox_ref[...], h_bf,
                             (((1,), (1,)), ((), ())),
                             preferred_element_type=jnp.float32)
        a = jax.nn.sigmoid(tx + tg[:, i:i + 1] + bco_ref[...])
        a = (a + sadj_ref[...]) * 0.5
        dv = lax.rsqrt(jnp.sum(a, axis=1, keepdims=True))
        m = (dv * h_bf.astype(jnp.float32)).astype(jnp.bfloat16)
        t_i = lax.dot_general(a.astype(jnp.bfloat16), m,
                              (((0,), (0,)), ((), ())),
                              preferred_element_type=jnp.float32)
        t_scr[i * n_nodes:(i + 1) * n_nodes, :] = \
            _leaky(dv * t_i).astype(jnp.bfloat16)
    z = _leaky(jnp.dot(t_scr[...], wdyn_ref[...],
                       preferred_element_type=jnp.float32))
    zw = z.reshape(BB, n_nodes, d) * wlast_ref[...][None]
    out2_ref[...] = jnp.sum(zw, axis=2)


def kernel(x_feat, static_adj, static_weight, dynamic_weight, w_fc, w_tr,
           b_tr, w_g, b_g, bn_gamma, bn_beta, w_co, b_co, w_last, b_last):
    B, Cf, H, W = x_feat.shape
    n_nodes = w_fc.shape[0]
    d = w_tr.shape[0]
    nblk = B // BB
    hw = H * W

    xt2 = x_feat.reshape(B, Cf, hw).astype(jnp.bfloat16)
    xt2 = xt2.transpose(0, 2, 1).reshape(B * hw, Cf)
    wfc_t = w_fc.T.astype(jnp.bfloat16)
    wtr_t = w_tr.T.astype(jnp.bfloat16)
    A = static_adj
    dvec = jnp.sum(A, axis=1) ** -0.5
    adjn = (dvec[:, None] * A.T * dvec[None, :]).astype(jnp.bfloat16)

    out1, h, y, o2a = pl.pallas_call(
        lambda *refs: _kern_a(*refs, n_nodes=n_nodes, d=d, hw=hw),
        grid=(nblk,),
        in_specs=[
            pl.BlockSpec((BB * hw, Cf), lambda i: (i, 0)),
            pl.BlockSpec((Cf, n_nodes), lambda i: (0, 0)),
            pl.BlockSpec((Cf, d), lambda i: (0, 0)),
            pl.BlockSpec((1, d), lambda i: (0, 0)),
            pl.BlockSpec((n_nodes, n_nodes), lambda i: (0, 0)),
            pl.BlockSpec((d, d), lambda i: (0, 0)),
            pl.BlockSpec((d, d), lambda i: (0, 0)),
            pl.BlockSpec((n_nodes, d), lambda i: (0, 0)),
        ],
        out_specs=[
            pl.BlockSpec((BB, n_nodes), lambda i: (i, 0)),
            pl.BlockSpec((BB * n_nodes, d), lambda i: (i, 0)),
            pl.BlockSpec((BB, d), lambda i: (i, 0)),
            pl.BlockSpec((BB * n_nodes, 1), lambda i: (i, 0)),
        ],
        out_shape=[
            jax.ShapeDtypeStruct((B, n_nodes), jnp.float32),
            jax.ShapeDtypeStruct((B * n_nodes, d), jnp.bfloat16),
            jax.ShapeDtypeStruct((B, d), jnp.float32),
            jax.ShapeDtypeStruct((B * n_nodes, 1), jnp.float32),
        ],
        scratch_shapes=[pltpu.VMEM((BB * n_nodes, d), jnp.bfloat16),
                        pltpu.VMEM((BB * n_nodes, d), jnp.bfloat16)],
        compiler_params=pltpu.CompilerParams(
            dimension_semantics=("parallel",)),
    )(xt2, wfc_t, wtr_t, b_tr.reshape(1, d), adjn,
      static_weight.astype(jnp.bfloat16), w_g.astype(jnp.bfloat16), w_last)

    y = y + b_g
    mu = jnp.mean(y, axis=0, keepdims=True)
    var = jnp.mean((y - mu) ** 2, axis=0, keepdims=True)
    g = _leaky((y - mu) * lax.rsqrt(var + BN_EPS) * bn_gamma + bn_beta)

    out2b = pl.pallas_call(
        lambda *refs: _kern_b(*refs, n_nodes=n_nodes, d=d),
        grid=(nblk,),
        in_specs=[
            pl.BlockSpec((BB * n_nodes, d), lambda i: (i, 0)),
            pl.BlockSpec((BB, d), lambda i: (i, 0)),
            pl.BlockSpec((n_nodes, d), lambda i: (0, 0)),
            pl.BlockSpec((n_nodes, d), lambda i: (0, 0)),
            pl.BlockSpec((n_nodes, 1), lambda i: (0, 0)),
            pl.BlockSpec((n_nodes, n_nodes), lambda i: (0, 0)),
            pl.BlockSpec((d, d), lambda i: (0, 0)),
            pl.BlockSpec((n_nodes, d), lambda i: (0, 0)),
        ],
        out_specs=pl.BlockSpec((BB, n_nodes), lambda i: (i, 0)),
        out_shape=jax.ShapeDtypeStruct((B, n_nodes), jnp.float32),
        scratch_shapes=[pltpu.VMEM((BB * n_nodes, d), jnp.bfloat16)],
        compiler_params=pltpu.CompilerParams(
            dimension_semantics=("parallel",)),
    )(h, g, w_co[:, :d].astype(jnp.bfloat16),
      w_co[:, d:].astype(jnp.bfloat16), b_co, static_adj,
      dynamic_weight.astype(jnp.bfloat16), w_last)
    out2 = out2b + o2a.reshape(B, n_nodes) + b_last.reshape(1, n_nodes)
    return out1, out2


# speedup vs baseline: 1.6203x; 1.0047x over previous
import jax
import jax.numpy as jnp
from jax import lax
from jax.experimental import pallas as pl
from jax.experimental.pallas import tpu as pltpu

NEG_SLOPE = 0.2
BN_EPS = 1e-5
BB = 16
ROWS = 56


def _leaky(x):
    return jnp.where(x >= 0, x, NEG_SLOPE * x)


def _kern_a(x_ref, wfc_ref, wtr_ref, btr_ref, adjn_ref, ws_ref, wg_ref,
            wlast_ref, out1_ref, h_ref, y_ref, o2a_ref, vt_scr, t_scr,
            *, n_nodes, d, hw):
    sc_all = jnp.dot(x_ref[...], wfc_ref[...],
                     preferred_element_type=jnp.float32)
    xt_all = jnp.dot(x_ref[...], wtr_ref[...],
                     preferred_element_type=jnp.float32) + btr_ref[...]
    rows = ROWS
    valid = lax.broadcasted_iota(jnp.int32, (rows, 1), 0) < hw
    for i in range(BB):
        sc = sc_all[i * rows:(i + 1) * rows, :]
        out1_ref[i:i + 1, :] = jnp.max(
            jnp.where(valid, sc, -jnp.inf), axis=0, keepdims=True)
        mask = jnp.where(valid, jax.nn.sigmoid(sc), 0.0)
        xt = xt_all[i * rows:(i + 1) * rows, :]
        vt_i = lax.dot_general(mask.astype(jnp.bfloat16),
                               xt.astype(jnp.bfloat16),
                               (((0,), (0,)), ((), ())),
                               preferred_element_type=jnp.float32)
        vt_scr[i * n_nodes:(i + 1) * n_nodes, :] = \
            vt_i.astype(jnp.bfloat16)
        o2a_ref[i * n_nodes:(i + 1) * n_nodes, :] = \
            jnp.sum(wlast_ref[...] * vt_i, axis=1, keepdims=True)
    v_bf = vt_scr[...]
    for i in range(BB):
        t_i = jnp.dot(adjn_ref[...],
                      v_bf[i * n_nodes:(i + 1) * n_nodes, :],
                      preferred_element_type=jnp.float32)
        t_scr[i * n_nodes:(i + 1) * n_nodes, :] = \
            _leaky(t_i).astype(jnp.bfloat16)
    h_all = v_bf.astype(jnp.float32) + jnp.dot(
        t_scr[...], ws_ref[...], preferred_element_type=jnp.float32)
    h_ref[...] = h_all.astype(jnp.bfloat16)
    xglb = jnp.mean(h_all.reshape(BB, n_nodes, d), axis=1)
    y_ref[...] = lax.dot_general(xglb.astype(jnp.bfloat16), wg_ref[...],
                                 (((1,), (1,)), ((), ())),
                                 preferred_element_type=jnp.float32)


def _kern_b(h_ref, g_ref, wcog_ref, wcox_ref, bco_ref, sadj_ref,
            wdyn_ref, wlast_ref, out2_ref, t_scr, *, n_nodes, d):
    tg = lax.dot_general(wcog_ref[...], g_ref[...].astype(jnp.bfloat16),
                         (((1,), (1,)), ((), ())),
                         preferred_element_type=jnp.float32)
    for i in range(BB):
        h_bf = h_ref[i * n_nodes:(i + 1) * n_nodes, :]
        tx = lax.dot_general(wcox_ref[...], h_bf,
                             (((1,), (1,)), ((), ())),
                             preferred_element_type=jnp.float32)
        a = jax.nn.sigmoid(tx + tg[:, i:i + 1] + bco_ref[...])
        a = (a + sadj_ref[...]) * 0.5
        dv = lax.rsqrt(jnp.sum(a, axis=1, keepdims=True))
        m = (dv * h_bf.astype(jnp.float32)).astype(jnp.bfloat16)
        t_i = lax.dot_general(a.astype(jnp.bfloat16), m,
                              (((0,), (0,)), ((), ())),
                              preferred_element_type=jnp.float32)
        t_scr[i * n_nodes:(i + 1) * n_nodes, :] = \
            _leaky(dv * t_i).astype(jnp.bfloat16)
    z = _leaky(jnp.dot(t_scr[...], wdyn_ref[...],
                       preferred_element_type=jnp.float32))
    zw = z.reshape(BB, n_nodes, d) * wlast_ref[...][None]
    out2_ref[...] = jnp.sum(zw, axis=2)


def kernel(x_feat, static_adj, static_weight, dynamic_weight, w_fc, w_tr,
           b_tr, w_g, b_g, bn_gamma, bn_beta, w_co, b_co, w_last, b_last):
    B, Cf, H, W = x_feat.shape
    n_nodes = w_fc.shape[0]
    d = w_tr.shape[0]
    nblk = B // BB
    hw = H * W

    xt2 = jnp.pad(x_feat.reshape(B, Cf, hw),
                  ((0, 0), (0, 0), (0, ROWS - hw))).astype(jnp.bfloat16)
    xt2 = xt2.transpose(0, 2, 1).reshape(B * ROWS, Cf)
    wfc_t = w_fc.T.astype(jnp.bfloat16)
    wtr_t = w_tr.T.astype(jnp.bfloat16)
    A = static_adj
    dvec = jnp.sum(A, axis=1) ** -0.5
    adjn = (dvec[:, None] * A.T * dvec[None, :]).astype(jnp.bfloat16)

    out1, h, y, o2a = pl.pallas_call(
        lambda *refs: _kern_a(*refs, n_nodes=n_nodes, d=d, hw=hw),
        grid=(nblk,),
        in_specs=[
            pl.BlockSpec((BB * ROWS, Cf), lambda i: (i, 0)),
            pl.BlockSpec((Cf, n_nodes), lambda i: (0, 0)),
            pl.BlockSpec((Cf, d), lambda i: (0, 0)),
            pl.BlockSpec((1, d), lambda i: (0, 0)),
            pl.BlockSpec((n_nodes, n_nodes), lambda i: (0, 0)),
            pl.BlockSpec((d, d), lambda i: (0, 0)),
            pl.BlockSpec((d, d), lambda i: (0, 0)),
            pl.BlockSpec((n_nodes, d), lambda i: (0, 0)),
        ],
        out_specs=[
            pl.BlockSpec((BB, n_nodes), lambda i: (i, 0)),
            pl.BlockSpec((BB * n_nodes, d), lambda i: (i, 0)),
            pl.BlockSpec((BB, d), lambda i: (i, 0)),
            pl.BlockSpec((BB * n_nodes, 1), lambda i: (i, 0)),
        ],
        out_shape=[
            jax.ShapeDtypeStruct((B, n_nodes), jnp.float32),
            jax.ShapeDtypeStruct((B * n_nodes, d), jnp.bfloat16),
            jax.ShapeDtypeStruct((B, d), jnp.float32),
            jax.ShapeDtypeStruct((B * n_nodes, 1), jnp.float32),
        ],
        scratch_shapes=[pltpu.VMEM((BB * n_nodes, d), jnp.bfloat16),
                        pltpu.VMEM((BB * n_nodes, d), jnp.bfloat16)],
        compiler_params=pltpu.CompilerParams(
            dimension_semantics=("parallel",)),
    )(xt2, wfc_t, wtr_t, b_tr.reshape(1, d), adjn,
      static_weight.astype(jnp.bfloat16), w_g.astype(jnp.bfloat16), w_last)

    y = y + b_g
    mu = jnp.mean(y, axis=0, keepdims=True)
    var = jnp.mean((y - mu) ** 2, axis=0, keepdims=True)
    g = _leaky((y - mu) * lax.rsqrt(var + BN_EPS) * bn_gamma + bn_beta)

    out2b = pl.pallas_call(
        lambda *refs: _kern_b(*refs, n_nodes=n_nodes, d=d),
        grid=(nblk,),
        in_specs=[
            pl.BlockSpec((BB * n_nodes, d), lambda i: (i, 0)),
            pl.BlockSpec((BB, d), lambda i: (i, 0)),
            pl.BlockSpec((n_nodes, d), lambda i: (0, 0)),
            pl.BlockSpec((n_nodes, d), lambda i: (0, 0)),
            pl.BlockSpec((n_nodes, 1), lambda i: (0, 0)),
            pl.BlockSpec((n_nodes, n_nodes), lambda i: (0, 0)),
            pl.BlockSpec((d, d), lambda i: (0, 0)),
            pl.BlockSpec((n_nodes, d), lambda i: (0, 0)),
        ],
        out_specs=pl.BlockSpec((BB, n_nodes), lambda i: (i, 0)),
        out_shape=jax.ShapeDtypeStruct((B, n_nodes), jnp.float32),
        scratch_shapes=[pltpu.VMEM((BB * n_nodes, d), jnp.bfloat16)],
        compiler_params=pltpu.CompilerParams(
            dimension_semantics=("parallel",)),
    )(h, g, w_co[:, :d].astype(jnp.bfloat16),
      w_co[:, d:].astype(jnp.bfloat16), b_co, static_adj,
      dynamic_weight.astype(jnp.bfloat16), w_last)
    out2 = out2b + o2a.reshape(B, n_nodes) + b_last.reshape(1, n_nodes)
    return out1, out2


# speedup vs baseline: 1.7076x; 1.0538x over previous
import jax
import jax.numpy as jnp
from jax import lax
from jax.experimental import pallas as pl
from jax.experimental.pallas import tpu as pltpu

NEG_SLOPE = 0.2
BN_EPS = 1e-5
BB = 16
ROWS = 56


def _leaky(x):
    return jnp.where(x >= 0, x, NEG_SLOPE * x)


def _kern_a(x_ref, wfc_ref, wtr_ref, btr_ref, adjn_ref, ws_ref, wg_ref,
            wlast_ref, out1_ref, h_ref, y_ref, o2a_ref, vt_scr, t_scr,
            *, n_nodes, d, hw):
    xb = x_ref[...].reshape(BB * ROWS, x_ref.shape[2])
    sc_all = jnp.dot(xb, wfc_ref[...],
                     preferred_element_type=jnp.float32)
    xt_all = jnp.dot(xb, wtr_ref[...],
                     preferred_element_type=jnp.float32) + btr_ref[...]
    rows = ROWS
    valid = lax.broadcasted_iota(jnp.int32, (rows, 1), 0) < hw
    for i in range(BB):
        sc = sc_all[i * rows:(i + 1) * rows, :]
        out1_ref[i:i + 1, :] = jnp.max(
            jnp.where(valid, sc, -jnp.inf), axis=0, keepdims=True)
        mask = jnp.where(valid, jax.nn.sigmoid(sc), 0.0)
        xt = xt_all[i * rows:(i + 1) * rows, :]
        vt_i = lax.dot_general(mask.astype(jnp.bfloat16),
                               xt.astype(jnp.bfloat16),
                               (((0,), (0,)), ((), ())),
                               preferred_element_type=jnp.float32)
        vt_scr[i * n_nodes:(i + 1) * n_nodes, :] = \
            vt_i.astype(jnp.bfloat16)
        o2a_ref[i * n_nodes:(i + 1) * n_nodes, :] = \
            jnp.sum(wlast_ref[...] * vt_i, axis=1, keepdims=True)
    v_bf = vt_scr[...]
    for i in range(BB):
        t_i = jnp.dot(adjn_ref[...],
                      v_bf[i * n_nodes:(i + 1) * n_nodes, :],
                      preferred_element_type=jnp.float32)
        t_scr[i * n_nodes:(i + 1) * n_nodes, :] = \
            _leaky(t_i).astype(jnp.bfloat16)
    h_all = v_bf.astype(jnp.float32) + jnp.dot(
        t_scr[...], ws_ref[...], preferred_element_type=jnp.float32)
    h_ref[...] = h_all.astype(jnp.bfloat16)
    xglb = jnp.mean(h_all.reshape(BB, n_nodes, d), axis=1)
    y_ref[...] = lax.dot_general(xglb.astype(jnp.bfloat16), wg_ref[...],
                                 (((1,), (1,)), ((), ())),
                                 preferred_element_type=jnp.float32)


def _kern_b(h_ref, g_ref, wcog_ref, wcox_ref, bco_ref, sadj_ref,
            wdyn_ref, wlast_ref, out2_ref, t_scr, *, n_nodes, d):
    tg = lax.dot_general(wcog_ref[...], g_ref[...].astype(jnp.bfloat16),
                         (((1,), (1,)), ((), ())),
                         preferred_element_type=jnp.float32)
    tx_all = lax.dot_general(wcox_ref[...], h_ref[...],
                             (((1,), (1,)), ((), ())),
                             preferred_element_type=jnp.float32)
    for i in range(BB):
        h_bf = h_ref[i * n_nodes:(i + 1) * n_nodes, :]
        tx = tx_all[:, i * n_nodes:(i + 1) * n_nodes]
        a = jax.nn.sigmoid(tx + tg[:, i:i + 1] + bco_ref[...])
        a = (a + sadj_ref[...]) * 0.5
        dv = lax.rsqrt(jnp.sum(a, axis=1, keepdims=True))
        m = (dv * h_bf.astype(jnp.float32)).astype(jnp.bfloat16)
        t_i = lax.dot_general(a.astype(jnp.bfloat16), m,
                              (((0,), (0,)), ((), ())),
                              preferred_element_type=jnp.float32)
        t_scr[i * n_nodes:(i + 1) * n_nodes, :] = \
            _leaky(dv * t_i).astype(jnp.bfloat16)
    z = _leaky(jnp.dot(t_scr[...], wdyn_ref[...],
                       preferred_element_type=jnp.float32))
    zw = z.reshape(BB, n_nodes, d) * wlast_ref[...][None]
    out2_ref[...] = jnp.sum(zw, axis=2)


def kernel(x_feat, static_adj, static_weight, dynamic_weight, w_fc, w_tr,
           b_tr, w_g, b_g, bn_gamma, bn_beta, w_co, b_co, w_last, b_last):
    B, Cf, H, W = x_feat.shape
    n_nodes = w_fc.shape[0]
    d = w_tr.shape[0]
    nblk = B // BB
    hw = H * W

    xt2 = jnp.pad(x_feat.reshape(B, Cf, hw),
                  ((0, 0), (0, 0), (0, ROWS - hw))).astype(jnp.bfloat16)
    xt2 = xt2.transpose(0, 2, 1)
    wfc_t = w_fc.T.astype(jnp.bfloat16)
    wtr_t = w_tr.T.astype(jnp.bfloat16)
    A = static_adj
    dvec = jnp.sum(A, axis=1) ** -0.5
    adjn = (dvec[:, None] * A.T * dvec[None, :]).astype(jnp.bfloat16)

    out1, h, y, o2a = pl.pallas_call(
        lambda *refs: _kern_a(*refs, n_nodes=n_nodes, d=d, hw=hw),
        grid=(nblk,),
        in_specs=[
            pl.BlockSpec((BB, ROWS, Cf), lambda i: (i, 0, 0)),
            pl.BlockSpec((Cf, n_nodes), lambda i: (0, 0)),
            pl.BlockSpec((Cf, d), lambda i: (0, 0)),
            pl.BlockSpec((1, d), lambda i: (0, 0)),
            pl.BlockSpec((n_nodes, n_nodes), lambda i: (0, 0)),
            pl.BlockSpec((d, d), lambda i: (0, 0)),
            pl.BlockSpec((d, d), lambda i: (0, 0)),
            pl.BlockSpec((n_nodes, d), lambda i: (0, 0)),
        ],
        out_specs=[
            pl.BlockSpec((BB, n_nodes), lambda i: (i, 0)),
            pl.BlockSpec((BB * n_nodes, d), lambda i: (i, 0)),
            pl.BlockSpec((BB, d), lambda i: (i, 0)),
            pl.BlockSpec((BB * n_nodes, 1), lambda i: (i, 0)),
        ],
        out_shape=[
            jax.ShapeDtypeStruct((B, n_nodes), jnp.float32),
            jax.ShapeDtypeStruct((B * n_nodes, d), jnp.bfloat16),
            jax.ShapeDtypeStruct((B, d), jnp.float32),
            jax.ShapeDtypeStruct((B * n_nodes, 1), jnp.float32),
        ],
        scratch_shapes=[pltpu.VMEM((BB * n_nodes, d), jnp.bfloat16),
                        pltpu.VMEM((BB * n_nodes, d), jnp.bfloat16)],
        compiler_params=pltpu.CompilerParams(
            dimension_semantics=("parallel",)),
    )(xt2, wfc_t, wtr_t, b_tr.reshape(1, d), adjn,
      static_weight.astype(jnp.bfloat16), w_g.astype(jnp.bfloat16), w_last)

    y = y + b_g
    mu = jnp.mean(y, axis=0, keepdims=True)
    var = jnp.mean((y - mu) ** 2, axis=0, keepdims=True)
    g = _leaky((y - mu) * lax.rsqrt(var + BN_EPS) * bn_gamma + bn_beta)

    out2b = pl.pallas_call(
        lambda *refs: _kern_b(*refs, n_nodes=n_nodes, d=d),
        grid=(nblk,),
        in_specs=[
            pl.BlockSpec((BB * n_nodes, d), lambda i: (i, 0)),
            pl.BlockSpec((BB, d), lambda i: (i, 0)),
            pl.BlockSpec((n_nodes, d), lambda i: (0, 0)),
            pl.BlockSpec((n_nodes, d), lambda i: (0, 0)),
            pl.BlockSpec((n_nodes, 1), lambda i: (0, 0)),
            pl.BlockSpec((n_nodes, n_nodes), lambda i: (0, 0)),
            pl.BlockSpec((d, d), lambda i: (0, 0)),
            pl.BlockSpec((n_nodes, d), lambda i: (0, 0)),
        ],
        out_specs=pl.BlockSpec((BB, n_nodes), lambda i: (i, 0)),
        out_shape=jax.ShapeDtypeStruct((B, n_nodes), jnp.float32),
        scratch_shapes=[pltpu.VMEM((BB * n_nodes, d), jnp.bfloat16)],
        compiler_params=pltpu.CompilerParams(
            dimension_semantics=("parallel",)),
    )(h, g, w_co[:, :d].astype(jnp.bfloat16),
      w_co[:, d:].astype(jnp.bfloat16), b_co, static_adj,
      dynamic_weight.astype(jnp.bfloat16), w_last)
    out2 = out2b + o2a.reshape(B, n_nodes) + b_last.reshape(1, n_nodes)
    return out1, out2
